# baseline (speedup 1.0000x reference)
import functools

import jax
import jax.numpy as jnp
from jax import lax
from jax.experimental import pallas as pl
from jax.experimental.pallas import tpu as pltpu
from jax.experimental.pallas import tpu_sc as plsc

BATCH = 4096
SEQ = 200
DIM = 64
TOTAL = BATCH * SEQ
VOCAB_ROWS = 1000002

A_BLK = 4096
A_HALF = A_BLK // 2
A_GRID = (VOCAB_ROWS + A_BLK - 1) // A_BLK
PACK_ROWS = A_GRID * A_HALF
TABLE_ROWS = 2 * PACK_ROWS


def _pack_body(tin, tout):
    t = tin[...].T
    tout[...] = jnp.concatenate([t[0:A_HALF], t[A_HALF:A_BLK]], axis=1)


def _pack_table(table_t):
    return pl.pallas_call(
        _pack_body,
        grid=(A_GRID,),
        in_specs=[pl.BlockSpec((DIM, A_BLK), lambda j: (0, j))],
        out_specs=pl.BlockSpec((A_HALF, 128), lambda j: (j, 0)),
        out_shape=jax.ShapeDtypeStruct((PACK_ROWS, 128), jnp.float32),
    )(table_t)


NUM_CORES = 2
NUM_SUBCORES = 16
NW = NUM_CORES * NUM_SUBCORES
B_PER_W = TOTAL // NW
NBUF = 3
CHUNK = 256
N_CHUNKS = B_PER_W // CHUNK
LOOKAHEAD = 2

_MESH = plsc.VectorSubcoreMesh(core_axis_name="c", subcore_axis_name="s")

_SCRATCH = (
    [pltpu.VMEM((CHUNK,), jnp.int32) for _ in range(NBUF)]
    + [pltpu.VMEM((CHUNK,), jnp.int32) for _ in range(NBUF)]
    + [pltpu.VMEM((CHUNK, DIM), jnp.float32) for _ in range(NBUF)]
    + [pltpu.VMEM((8, 2, 8, 128), jnp.float32) for _ in range(NBUF)]
    + [pltpu.SemaphoreType.DMA for _ in range(2 * NBUF)]
)

@functools.partial(
    pl.kernel,
    mesh=_MESH,
    out_type=jax.ShapeDtypeStruct((SEQ, 8, BATCH // 128, 8, 128), jnp.float32),
    scratch_types=_SCRATCH,
    compiler_params=pltpu.CompilerParams(
        use_tc_tiling_on_sc=False, needs_layout_passes=False
    ),
)
def _gather_rows(tok_hbm, table_hbm, out_hbm, *refs):
    tok_v = refs[0:NBUF]
    idx_v = refs[NBUF : 2 * NBUF]
    rows_v = refs[2 * NBUF : 3 * NBUF]
    st_v = refs[3 * NBUF : 4 * NBUF]
    sg = refs[4 * NBUF : 5 * NBUF]
    ss = refs[5 * NBUF : 6 * NBUF]

    wid = lax.axis_index("s") * NUM_CORES + lax.axis_index("c")
    base = wid * B_PER_W

    def launch(i, b):
        off = base + i * CHUNK
        pltpu.sync_copy(tok_hbm.at[pl.ds(off, CHUNK)], tok_v[b])
        for k in range(CHUNK // 16):
            t = tok_v[b][pl.ds(16 * k, 16)]
            p = ((t >> 12) << 12) | ((t & 2047) << 1) | ((t >> 11) & 1)
            idx_v[b][pl.ds(16 * k, 16)] = p
        pltpu.async_copy(table_hbm.at[idx_v[b]], rows_v[b], sg[b])

    def gather_wait(b):
        pltpu.make_async_copy(table_hbm.at[idx_v[b]], rows_v[b], sg[b]).wait()

    lane = lax.iota(jnp.int32, 16)
    jt_ks = [(16 * k + lane) >> 3 for k in range(4)]
    r_ks = [(16 * k + lane) & 7 for k in range(4)]

    def transpose(b):
        def trow(t, carry):
            bti = jnp.zeros((16,), jnp.int32) + ((t >> 7) & 1)
            lpos = jnp.zeros((16,), jnp.int32) + (t & 127)
            for k in range(4):
                v = rows_v[b][t, pl.ds(16 * k, 16)]
                plsc.store_scatter(st_v[b], [jt_ks[k], bti, r_ks[k], lpos], v)
            return carry

        lax.fori_loop(0, CHUNK, trow, 0)

    def store_start(i, b):
        t0 = base + i * CHUNK
        s = t0 >> 12
        bt0 = (t0 & 4095) >> 7
        for jt in range(8):
            pltpu.async_copy(
                st_v[b].at[jt],
                out_hbm.at[s, jt, pl.ds(bt0, 2), :, :],
                ss[b],
            )

    def store_wait(b):
        for jt in range(8):
            pltpu.make_async_copy(
                st_v[b].at[jt],
                out_hbm.at[0, 0, pl.ds(0, 2), :, :],
                ss[b],
            ).wait()

    for i in range(LOOKAHEAD):
        launch(i, i % NBUF)

    for i in range(NBUF):
        b = i % NBUF
        gather_wait(b)
        transpose(b)
        store_start(i, b)
        if i + LOOKAHEAD < N_CHUNKS:
            launch(i + LOOKAHEAD, (i + LOOKAHEAD) % NBUF)

    def outer(g, carry):
        for bo in range(NBUF):
            i = g * NBUF + bo
            b = bo
            gather_wait(b)
            store_wait(b)
            transpose(b)
            store_start(i, b)
            ni = i + LOOKAHEAD

            @pl.when(ni < N_CHUNKS)
            def _():
                launch(ni, (bo + LOOKAHEAD) % NBUF)

        return carry

    lax.fori_loop(1, (N_CHUNKS - 1) // NBUF, outer, 0)

    i_last = N_CHUNKS - 1
    b_last = i_last % NBUF
    gather_wait(b_last)
    store_wait(b_last)
    transpose(b_last)
    store_start(i_last, b_last)

    for b in range(NBUF):
        store_wait(b)


def kernel(token_seqs, emb_table):
    table_t = emb_table.T
    table_lin = _pack_table(table_t).reshape(TABLE_ROWS, DIM)
    toks = token_seqs.T.reshape(-1).astype(jnp.int32)
    t5 = _gather_rows(toks, table_lin)
    return t5.transpose(2, 4, 0, 1, 3).reshape(BATCH, SEQ, DIM)


# speedup vs baseline: 1.9535x; 1.9535x over previous
import functools

import jax
import jax.numpy as jnp
from jax import lax
from jax.experimental import pallas as pl
from jax.experimental.pallas import tpu as pltpu
from jax.experimental.pallas import tpu_sc as plsc

BATCH = 4096
SEQ = 200
DIM = 64
TOTAL = BATCH * SEQ
VOCAB_ROWS = 1000002

A_BLK = 4096
A_HALF = A_BLK // 2
A_GRID = (VOCAB_ROWS + A_BLK - 1) // A_BLK
PACK_ROWS = A_GRID * A_HALF
TABLE_ROWS = 2 * PACK_ROWS


def _pack_body(tin, tout):
    t = tin[...].T
    tout[...] = jnp.concatenate([t[0:A_HALF], t[A_HALF:A_BLK]], axis=1)


def _pack_table(table_t):
    return pl.pallas_call(
        _pack_body,
        grid=(A_GRID,),
        in_specs=[pl.BlockSpec((DIM, A_BLK), lambda j: (0, j))],
        out_specs=pl.BlockSpec((A_HALF, 128), lambda j: (j, 0)),
        out_shape=jax.ShapeDtypeStruct((PACK_ROWS, 128), jnp.float32),
    )(table_t)


NUM_CORES = 2
NUM_SUBCORES = 16
NW = NUM_CORES * NUM_SUBCORES
B_PER_W = TOTAL // NW
NBUF = 4
CHUNK = 256
N_CHUNKS = B_PER_W // CHUNK
LOOKAHEAD = 2
N_OUTER = N_CHUNKS // NBUF
OUT_ROWS = TOTAL // 2

_MESH = plsc.VectorSubcoreMesh(core_axis_name="c", subcore_axis_name="s")

_SCRATCH = (
    [pltpu.VMEM((CHUNK,), jnp.int32) for _ in range(NBUF)]
    + [pltpu.VMEM((CHUNK,), jnp.int32) for _ in range(NBUF)]
    + [pltpu.VMEM((CHUNK, DIM), jnp.float32) for _ in range(NBUF)]
    + [pltpu.SemaphoreType.DMA for _ in range(2 * NBUF)]
)


@functools.partial(
    pl.kernel,
    mesh=_MESH,
    out_type=jax.ShapeDtypeStruct((OUT_ROWS, 128), jnp.float32),
    scratch_types=_SCRATCH,
    compiler_params=pltpu.CompilerParams(use_tc_tiling_on_sc=False),
)
def _gather_rows(tok_hbm, table_hbm, out_hbm, *refs):
    tok_v = refs[0:NBUF]
    idx_v = refs[NBUF : 2 * NBUF]
    rows_v = refs[2 * NBUF : 3 * NBUF]
    sg = refs[3 * NBUF : 4 * NBUF]
    ss = refs[4 * NBUF : 5 * NBUF]

    wid = lax.axis_index("s") * NUM_CORES + lax.axis_index("c")
    base = wid * B_PER_W

    def launch(i, b):
        off = base + i * CHUNK
        pltpu.sync_copy(tok_hbm.at[pl.ds(off, CHUNK)], tok_v[b])
        for k in range(CHUNK // 16):
            t = tok_v[b][pl.ds(16 * k, 16)]
            p = ((t >> 12) << 12) | ((t & 2047) << 1) | ((t >> 11) & 1)
            idx_v[b][pl.ds(16 * k, 16)] = p
        pltpu.async_copy(table_hbm.at[idx_v[b]], rows_v[b], sg[b])

    def gather_wait(b):
        pltpu.make_async_copy(table_hbm.at[idx_v[b]], rows_v[b], sg[b]).wait()

    def store_slot(i):
        t0 = base + i * CHUNK
        q0 = t0 & 2047
        r0 = (t0 >> 11) * 1024 + (q0 & 1023)
        h = q0 >> 10
        return r0, h

    def store_start(i, b):
        r0, h = store_slot(i)
        pltpu.async_copy(
            rows_v[b], out_hbm.at[pl.ds(r0, CHUNK), pl.ds(h * DIM, DIM)], ss[b]
        )

    def store_wait(b):
        pltpu.make_async_copy(
            rows_v[b], out_hbm.at[pl.ds(base // 2, CHUNK), pl.ds(0, DIM)], ss[b]
        ).wait()

    for i in range(LOOKAHEAD):
        launch(i, i % NBUF)

    for b in range(NBUF):
        gather_wait(b)
        store_start(b, b)
        ni = b + LOOKAHEAD
        nb = ni % NBUF
        if ni < NBUF:
            launch(ni, nb)
        else:
            store_wait(nb)
            launch(ni, nb)

    def outer(g, carry):
        for b in range(NBUF):
            i = g * NBUF + b
            gather_wait(b)
            store_start(i, b)
            ni = i + LOOKAHEAD
            nb = (b + LOOKAHEAD) % NBUF

            @pl.when(ni < N_CHUNKS)
            def _():
                store_wait(nb)
                launch(ni, nb)

        return carry

    lax.fori_loop(1, N_OUTER, outer, 0)

    for b in range(NBUF):
        store_wait(b)


def _unpack_body(rin, gout):
    xa = rin[0].T
    xb = rin[1].T
    gout[...] = jnp.concatenate(
        [xa[0:DIM], xa[DIM : 2 * DIM], xb[0:DIM], xb[DIM : 2 * DIM]], axis=1
    )[None]


def _unpack_out(rows3):
    return pl.pallas_call(
        _unpack_body,
        grid=(SEQ,),
        in_specs=[pl.BlockSpec((2, 1024, 128), lambda s: (s, 0, 0))],
        out_specs=pl.BlockSpec((1, DIM, BATCH), lambda s: (s, 0, 0)),
        out_shape=jax.ShapeDtypeStruct((SEQ, DIM, BATCH), jnp.float32),
    )(rows3)


def kernel(token_seqs, emb_table):
    table_t = emb_table.T
    table_lin = _pack_table(table_t).reshape(TABLE_ROWS, DIM)
    toks = token_seqs.T.reshape(-1).astype(jnp.int32)
    rows = _gather_rows(toks, table_lin)
    rows3 = rows.reshape(TOTAL // 2048, 1024, 128)
    g = _unpack_out(rows3)
    return g.transpose(2, 0, 1)


# speedup vs baseline: 2.1630x; 1.1073x over previous
import functools

import jax
import jax.numpy as jnp
from jax import lax
from jax.experimental import pallas as pl
from jax.experimental.pallas import tpu as pltpu
from jax.experimental.pallas import tpu_sc as plsc

BATCH = 4096
SEQ = 200
DIM = 64
TOTAL = BATCH * SEQ
VOCAB_ROWS = 1000002

A_BLK = 8192
A_HALF = A_BLK // 2
A_GRID = (VOCAB_ROWS + A_BLK - 1) // A_BLK
PACK_ROWS = A_GRID * A_HALF
TABLE_ROWS = 2 * PACK_ROWS


def _pack_body(tin, tout):
    t = tin[...].T
    tout[...] = jnp.concatenate([t[0:A_HALF], t[A_HALF:A_BLK]], axis=1)


def _pack_table(table_t):
    return pl.pallas_call(
        _pack_body,
        grid=(A_GRID,),
        in_specs=[pl.BlockSpec((DIM, A_BLK), lambda j: (0, j))],
        out_specs=pl.BlockSpec((A_HALF, 128), lambda j: (j, 0)),
        out_shape=jax.ShapeDtypeStruct((PACK_ROWS, 128), jnp.float32),
    )(table_t)


NUM_CORES = 2
NUM_SUBCORES = 16
NW = NUM_CORES * NUM_SUBCORES
B_PER_W = TOTAL // NW
NBUF = 4
CHUNK = 256
N_CHUNKS = B_PER_W // CHUNK
LOOKAHEAD = 2
N_OUTER = N_CHUNKS // NBUF
OUT_ROWS = TOTAL // 2

_MESH = plsc.VectorSubcoreMesh(core_axis_name="c", subcore_axis_name="s")

_SCRATCH = (
    [pltpu.VMEM((CHUNK,), jnp.int32) for _ in range(NBUF)]
    + [pltpu.VMEM((CHUNK,), jnp.int32) for _ in range(NBUF)]
    + [pltpu.VMEM((CHUNK, DIM), jnp.float32) for _ in range(NBUF)]
    + [pltpu.SemaphoreType.DMA for _ in range(2 * NBUF)]
)


@functools.partial(
    pl.kernel,
    mesh=_MESH,
    out_type=jax.ShapeDtypeStruct((OUT_ROWS, 128), jnp.float32),
    scratch_types=_SCRATCH,
    compiler_params=pltpu.CompilerParams(use_tc_tiling_on_sc=False),
)
def _gather_rows(tok_hbm, table_hbm, out_hbm, *refs):
    tok_v = refs[0:NBUF]
    idx_v = refs[NBUF : 2 * NBUF]
    rows_v = refs[2 * NBUF : 3 * NBUF]
    sg = refs[3 * NBUF : 4 * NBUF]
    ss = refs[4 * NBUF : 5 * NBUF]

    wid = lax.axis_index("s") * NUM_CORES + lax.axis_index("c")
    base = wid * B_PER_W

    def launch(i, b):
        off = base + i * CHUNK
        pltpu.sync_copy(tok_hbm.at[pl.ds(off, CHUNK)], tok_v[b])
        for k in range(CHUNK // 16):
            t = tok_v[b][pl.ds(16 * k, 16)]
            p = ((t >> 13) << 13) | ((t & 4095) << 1) | ((t >> 12) & 1)
            idx_v[b][pl.ds(16 * k, 16)] = p
        pltpu.async_copy(table_hbm.at[idx_v[b]], rows_v[b], sg[b])

    def gather_wait(b):
        pltpu.make_async_copy(table_hbm.at[idx_v[b]], rows_v[b], sg[b]).wait()

    def store_slot(i):
        t0 = base + i * CHUNK
        q0 = t0 & 2047
        r0 = (t0 >> 11) * 1024 + (q0 & 1023)
        h = q0 >> 10
        return r0, h

    def store_start(i, b):
        r0, h = store_slot(i)
        pltpu.async_copy(
            rows_v[b], out_hbm.at[pl.ds(r0, CHUNK), pl.ds(h * DIM, DIM)], ss[b]
        )

    def store_wait(b):
        pltpu.make_async_copy(
            rows_v[b], out_hbm.at[pl.ds(base // 2, CHUNK), pl.ds(0, DIM)], ss[b]
        ).wait()

    for i in range(LOOKAHEAD):
        launch(i, i % NBUF)

    for b in range(NBUF):
        gather_wait(b)
        store_start(b, b)
        ni = b + LOOKAHEAD
        nb = ni % NBUF
        if ni < NBUF:
            launch(ni, nb)
        else:
            store_wait(nb)
            launch(ni, nb)

    def outer(g, carry):
        for b in range(NBUF):
            i = g * NBUF + b
            gather_wait(b)
            store_start(i, b)
            ni = i + LOOKAHEAD
            nb = (b + LOOKAHEAD) % NBUF

            @pl.when(ni < N_CHUNKS)
            def _():
                store_wait(nb)
                launch(ni, nb)

        return carry

    lax.fori_loop(1, N_OUTER, outer, 0)

    for b in range(NBUF):
        store_wait(b)


def _unpack_body(rin, gout):
    xa = rin[0].T
    xb = rin[1].T
    gout[...] = jnp.concatenate(
        [xa[0:DIM], xa[DIM : 2 * DIM], xb[0:DIM], xb[DIM : 2 * DIM]], axis=1
    )[None]


def _unpack_out(rows3):
    return pl.pallas_call(
        _unpack_body,
        grid=(SEQ,),
        in_specs=[pl.BlockSpec((2, 1024, 128), lambda s: (s, 0, 0))],
        out_specs=pl.BlockSpec((1, DIM, BATCH), lambda s: (s, 0, 0)),
        out_shape=jax.ShapeDtypeStruct((SEQ, DIM, BATCH), jnp.float32),
    )(rows3)


def kernel(token_seqs, emb_table):
    table_t = emb_table.T
    table_lin = _pack_table(table_t).reshape(TABLE_ROWS, DIM)
    toks = token_seqs.T.reshape(-1).astype(jnp.int32)
    rows = _gather_rows(toks, table_lin)
    rows3 = rows.reshape(TOTAL // 2048, 1024, 128)
    g = _unpack_out(rows3)
    return g.transpose(2, 0, 1)


# speedup vs baseline: 2.5362x; 1.1726x over previous
import functools

import jax
import jax.numpy as jnp
from jax import lax
from jax.experimental import pallas as pl
from jax.experimental.pallas import tpu as pltpu
from jax.experimental.pallas import tpu_sc as plsc

BATCH = 4096
SEQ = 200
DIM = 64
TOTAL = BATCH * SEQ
VOCAB_ROWS = 1000002

A_BLK = 16384
A_HALF = A_BLK // 2
A_GRID = (VOCAB_ROWS + A_BLK - 1) // A_BLK
PACK_ROWS = A_GRID * A_HALF
TABLE_ROWS = 2 * PACK_ROWS


def _pack_body(tin, tout):
    t = tin[...].T
    tout[...] = jnp.concatenate([t[0:A_HALF], t[A_HALF:A_BLK]], axis=1)


def _pack_table(table_t):
    return pl.pallas_call(
        _pack_body,
        grid=(A_GRID,),
        in_specs=[pl.BlockSpec((DIM, A_BLK), lambda j: (0, j))],
        out_specs=pl.BlockSpec((A_HALF, 128), lambda j: (j, 0)),
        out_shape=jax.ShapeDtypeStruct((PACK_ROWS, 128), jnp.float32),
    )(table_t)


NUM_CORES = 2
NUM_SUBCORES = 16
NW = NUM_CORES * NUM_SUBCORES
B_PER_W = TOTAL // NW
NBUF = 4
CHUNK = 256
N_CHUNKS = B_PER_W // CHUNK
LOOKAHEAD = 2
N_OUTER = N_CHUNKS // NBUF
OUT_ROWS = TOTAL // 2

_MESH = plsc.VectorSubcoreMesh(core_axis_name="c", subcore_axis_name="s")

_SCRATCH = (
    [pltpu.VMEM((CHUNK,), jnp.int32) for _ in range(NBUF)]
    + [pltpu.VMEM((CHUNK,), jnp.int32) for _ in range(NBUF)]
    + [pltpu.VMEM((CHUNK, DIM), jnp.float32) for _ in range(NBUF)]
    + [pltpu.SemaphoreType.DMA for _ in range(2 * NBUF)]
)


@functools.partial(
    pl.kernel,
    mesh=_MESH,
    out_type=jax.ShapeDtypeStruct((OUT_ROWS, 128), jnp.float32),
    scratch_types=_SCRATCH,
    compiler_params=pltpu.CompilerParams(use_tc_tiling_on_sc=False),
)
def _gather_rows(tok_hbm, table_hbm, out_hbm, *refs):
    tok_v = refs[0:NBUF]
    idx_v = refs[NBUF : 2 * NBUF]
    rows_v = refs[2 * NBUF : 3 * NBUF]
    sg = refs[3 * NBUF : 4 * NBUF]
    ss = refs[4 * NBUF : 5 * NBUF]

    wid = lax.axis_index("s") * NUM_CORES + lax.axis_index("c")
    base = wid * B_PER_W

    def launch(i, b):
        off = base + i * CHUNK
        pltpu.sync_copy(tok_hbm.at[pl.ds(off, CHUNK)], tok_v[b])
        for k in range(CHUNK // 16):
            t = tok_v[b][pl.ds(16 * k, 16)]
            p = ((t >> 14) << 14) | ((t & 8191) << 1) | ((t >> 13) & 1)
            idx_v[b][pl.ds(16 * k, 16)] = p
        pltpu.async_copy(table_hbm.at[idx_v[b]], rows_v[b], sg[b])

    def gather_wait(b):
        pltpu.make_async_copy(table_hbm.at[idx_v[b]], rows_v[b], sg[b]).wait()

    def store_slot(i):
        t0 = base + i * CHUNK
        q0 = t0 & 2047
        r0 = (t0 >> 11) * 1024 + (q0 & 1023)
        h = q0 >> 10
        return r0, h

    def store_start(i, b):
        r0, h = store_slot(i)
        pltpu.async_copy(
            rows_v[b], out_hbm.at[pl.ds(r0, CHUNK), pl.ds(h * DIM, DIM)], ss[b]
        )

    def store_wait(b):
        pltpu.make_async_copy(
            rows_v[b], out_hbm.at[pl.ds(base // 2, CHUNK), pl.ds(0, DIM)], ss[b]
        ).wait()

    for i in range(LOOKAHEAD):
        launch(i, i % NBUF)

    for b in range(NBUF):
        gather_wait(b)
        store_start(b, b)
        ni = b + LOOKAHEAD
        nb = ni % NBUF
        if ni < NBUF:
            launch(ni, nb)
        else:
            store_wait(nb)
            launch(ni, nb)

    def outer(g, carry):
        for b in range(NBUF):
            i = g * NBUF + b
            gather_wait(b)
            store_start(i, b)
            ni = i + LOOKAHEAD
            nb = (b + LOOKAHEAD) % NBUF

            @pl.when(ni < N_CHUNKS)
            def _():
                store_wait(nb)
                launch(ni, nb)

        return carry

    lax.fori_loop(1, N_OUTER, outer, 0)

    for b in range(NBUF):
        store_wait(b)


def _unpack_body(rin, gout):
    for i in range(2):
        xa = rin[2 * i].T
        xb = rin[2 * i + 1].T
        gout[i] = jnp.concatenate(
            [xa[0:DIM], xa[DIM : 2 * DIM], xb[0:DIM], xb[DIM : 2 * DIM]],
            axis=1,
        )


def _unpack_out(rows3):
    return pl.pallas_call(
        _unpack_body,
        grid=(SEQ // 2,),
        in_specs=[pl.BlockSpec((4, 1024, 128), lambda s: (s, 0, 0))],
        out_specs=pl.BlockSpec((2, DIM, BATCH), lambda s: (s, 0, 0)),
        out_shape=jax.ShapeDtypeStruct((SEQ, DIM, BATCH), jnp.float32),
    )(rows3)


def kernel(token_seqs, emb_table):
    table_t = emb_table.T
    table_lin = _pack_table(table_t).reshape(TABLE_ROWS, DIM)
    toks = token_seqs.T.reshape(-1).astype(jnp.int32)
    rows = _gather_rows(toks, table_lin)
    rows3 = rows.reshape(TOTAL // 2048, 1024, 128)
    g = _unpack_out(rows3)
    return g.transpose(2, 0, 1)


# speedup vs baseline: 2.7324x; 1.0774x over previous
import functools

import jax
import jax.numpy as jnp
from jax import lax
from jax.experimental import pallas as pl
from jax.experimental.pallas import tpu as pltpu
from jax.experimental.pallas import tpu_sc as plsc

BATCH = 4096
SEQ = 200
DIM = 64
TOTAL = BATCH * SEQ
VOCAB_ROWS = 1000002

A_BLK = 32768
A_HALF = A_BLK // 2
A_GRID = (VOCAB_ROWS + A_BLK - 1) // A_BLK
PACK_ROWS = A_GRID * A_HALF
TABLE_ROWS = 2 * PACK_ROWS


def _pack_body(tin, tout):
    t = tin[...].T
    tout[...] = jnp.concatenate([t[0:A_HALF], t[A_HALF:A_BLK]], axis=1)


def _pack_table(table_t):
    return pl.pallas_call(
        _pack_body,
        grid=(A_GRID,),
        in_specs=[pl.BlockSpec((DIM, A_BLK), lambda j: (0, j))],
        out_specs=pl.BlockSpec((A_HALF, 128), lambda j: (j, 0)),
        out_shape=jax.ShapeDtypeStruct((PACK_ROWS, 128), jnp.float32),
    )(table_t)


NUM_CORES = 2
NUM_SUBCORES = 16
NW = NUM_CORES * NUM_SUBCORES
B_PER_W = TOTAL // NW
NBUF = 4
CHUNK = 256
N_CHUNKS = B_PER_W // CHUNK
LOOKAHEAD = 2
N_OUTER = N_CHUNKS // NBUF
OUT_ROWS = TOTAL // 2

_MESH = plsc.VectorSubcoreMesh(core_axis_name="c", subcore_axis_name="s")

_SCRATCH = (
    [pltpu.VMEM((CHUNK,), jnp.int32) for _ in range(NBUF)]
    + [pltpu.VMEM((CHUNK,), jnp.int32) for _ in range(NBUF)]
    + [pltpu.VMEM((CHUNK, DIM), jnp.float32) for _ in range(NBUF)]
    + [pltpu.SemaphoreType.DMA for _ in range(2 * NBUF)]
)


@functools.partial(
    pl.kernel,
    mesh=_MESH,
    out_type=jax.ShapeDtypeStruct((OUT_ROWS, 128), jnp.float32),
    scratch_types=_SCRATCH,
    compiler_params=pltpu.CompilerParams(use_tc_tiling_on_sc=False),
)
def _gather_rows(tok_hbm, table_hbm, out_hbm, *refs):
    tok_v = refs[0:NBUF]
    idx_v = refs[NBUF : 2 * NBUF]
    rows_v = refs[2 * NBUF : 3 * NBUF]
    sg = refs[3 * NBUF : 4 * NBUF]
    ss = refs[4 * NBUF : 5 * NBUF]

    wid = lax.axis_index("s") * NUM_CORES + lax.axis_index("c")
    base = wid * B_PER_W

    def launch(i, b):
        off = base + i * CHUNK
        pltpu.sync_copy(tok_hbm.at[pl.ds(off, CHUNK)], tok_v[b])
        for k in range(CHUNK // 16):
            t = tok_v[b][pl.ds(16 * k, 16)]
            p = ((t >> 15) << 15) | ((t & 16383) << 1) | ((t >> 14) & 1)
            idx_v[b][pl.ds(16 * k, 16)] = p
        pltpu.async_copy(table_hbm.at[idx_v[b]], rows_v[b], sg[b])

    def gather_wait(b):
        pltpu.make_async_copy(table_hbm.at[idx_v[b]], rows_v[b], sg[b]).wait()

    def store_slot(i):
        t0 = base + i * CHUNK
        q0 = t0 & 2047
        r0 = (t0 >> 11) * 1024 + (q0 & 1023)
        h = q0 >> 10
        return r0, h

    def store_start(i, b):
        r0, h = store_slot(i)
        pltpu.async_copy(
            rows_v[b], out_hbm.at[pl.ds(r0, CHUNK), pl.ds(h * DIM, DIM)], ss[b]
        )

    def store_wait(b):
        pltpu.make_async_copy(
            rows_v[b], out_hbm.at[pl.ds(base // 2, CHUNK), pl.ds(0, DIM)], ss[b]
        ).wait()

    for i in range(LOOKAHEAD):
        launch(i, i % NBUF)

    for b in range(NBUF):
        gather_wait(b)
        store_start(b, b)
        ni = b + LOOKAHEAD
        nb = ni % NBUF
        if ni < NBUF:
            launch(ni, nb)
        else:
            store_wait(nb)
            launch(ni, nb)

    def outer(g, carry):
        for b in range(NBUF):
            i = g * NBUF + b
            gather_wait(b)
            store_start(i, b)
            ni = i + LOOKAHEAD
            nb = (b + LOOKAHEAD) % NBUF

            @pl.when(ni < N_CHUNKS)
            def _():
                store_wait(nb)
                launch(ni, nb)

        return carry

    lax.fori_loop(1, N_OUTER, outer, 0)

    for b in range(NBUF):
        store_wait(b)


def _unpack_body(rin, gout):
    for i in range(4):
        xa = rin[2 * i].T
        xb = rin[2 * i + 1].T
        gout[i] = jnp.concatenate(
            [xa[0:DIM], xa[DIM : 2 * DIM], xb[0:DIM], xb[DIM : 2 * DIM]],
            axis=1,
        )


def _unpack_out(rows3):
    return pl.pallas_call(
        _unpack_body,
        grid=(SEQ // 4,),
        in_specs=[pl.BlockSpec((8, 1024, 128), lambda s: (s, 0, 0))],
        out_specs=pl.BlockSpec((4, DIM, BATCH), lambda s: (s, 0, 0)),
        out_shape=jax.ShapeDtypeStruct((SEQ, DIM, BATCH), jnp.float32),
    )(rows3)


def kernel(token_seqs, emb_table):
    table_t = emb_table.T
    table_lin = _pack_table(table_t).reshape(TABLE_ROWS, DIM)
    toks = token_seqs.T.reshape(-1).astype(jnp.int32)
    rows = _gather_rows(toks, table_lin)
    rows3 = rows.reshape(TOTAL // 2048, 1024, 128)
    g = _unpack_out(rows3)
    return g.transpose(2, 0, 1)
